# Initial kernel scaffold; baseline (speedup 1.0000x reference)
import jax
import jax.numpy as jnp
from jax.experimental import pallas as pl


def kernel(x, edge_index, Wl1, Wr1, b1, Wl2, Wr2, b2, Wg, att_src, att_dst, bg):
    raise NotImplementedError("write your pallas kernel here")


# speedup vs baseline: 23.5685x; 23.5685x over previous
import functools

import jax
import jax.numpy as jnp
from jax import lax
from jax.experimental import pallas as pl
from jax.experimental.pallas import tpu as pltpu
from jax.experimental.pallas import tpu_sc as plsc

N = 10000
E = 320000
D = 128
NC = 2
NS = 16
NW = NC * NS
EW = E // NW
GSZ = 80
NG = EW // GSZ
RPT = 624
TAIL = N - RPT * NS
BLK = 1000
GRID = N // BLK

_f32 = jnp.float32
_CONTRACT_11 = (((1,), (1,)), ((), ()))


def _lin2_body(x_ref, wl_ref, wr_ref, b_ref, y_ref, r_ref):
    x = x_ref[...]
    y_ref[...] = lax.dot_general(x, wl_ref[...], _CONTRACT_11,
                                 preferred_element_type=_f32)
    r_ref[...] = lax.dot_general(x, wr_ref[...], _CONTRACT_11,
                                 preferred_element_type=_f32) + b_ref[...]


def _lin2(x, wl, wr, b):
    return pl.pallas_call(
        _lin2_body,
        grid=(GRID,),
        in_specs=[
            pl.BlockSpec((BLK, D), lambda i: (i, 0)),
            pl.BlockSpec((D, D), lambda i: (0, 0)),
            pl.BlockSpec((D, D), lambda i: (0, 0)),
            pl.BlockSpec((1, D), lambda i: (0, 0)),
        ],
        out_specs=[pl.BlockSpec((BLK, D), lambda i: (i, 0))] * 2,
        out_shape=[jax.ShapeDtypeStruct((N, D), _f32)] * 2,
    )(x, wl, wr, b.reshape(1, D))


def _sage_next_body(s_ref, dg_ref, r_ref, wl_ref, wr_ref, b_ref, z_ref, r2_ref):
    deg = dg_ref[0, 0, 0, :] + dg_ref[1, 0, 0, :]
    rec = 1.0 / jnp.maximum(deg, 1.0)
    agg = (s_ref[0] + s_ref[1]) * rec[:, None]
    h = jnp.maximum(agg + r_ref[...], 0.0)
    z_ref[...] = lax.dot_general(h, wl_ref[...], _CONTRACT_11,
                                 preferred_element_type=_f32)
    r2_ref[...] = lax.dot_general(h, wr_ref[...], _CONTRACT_11,
                                  preferred_element_type=_f32) + b_ref[...]


def _sage_next(ssum, deg3, r, wl, wr, b):
    return pl.pallas_call(
        _sage_next_body,
        grid=(GRID,),
        in_specs=[
            pl.BlockSpec((NC, BLK, D), lambda i: (0, i, 0)),
            pl.BlockSpec((NC, 1, 1, BLK), lambda i: (0, i, 0, 0)),
            pl.BlockSpec((BLK, D), lambda i: (i, 0)),
            pl.BlockSpec((D, D), lambda i: (0, 0)),
            pl.BlockSpec((D, D), lambda i: (0, 0)),
            pl.BlockSpec((1, D), lambda i: (0, 0)),
        ],
        out_specs=[pl.BlockSpec((BLK, D), lambda i: (i, 0))] * 2,
        out_shape=[jax.ShapeDtypeStruct((N, D), _f32)] * 2,
    )(ssum, deg3, r, wl, wr, b.reshape(1, D))


def _wc_body(wg_ref, as_ref, ad_ref, wc_ref):
    wg = wg_ref[...]
    vs = jnp.dot(as_ref[...], wg, preferred_element_type=_f32)
    vd = jnp.dot(ad_ref[...], wg, preferred_element_type=_f32)
    wc_ref[...] = jnp.concatenate(
        [wg, vs, vd, jnp.zeros((4, D), _f32)], axis=0)


def _wc(wg, att_s, att_d):
    return pl.pallas_call(
        _wc_body,
        out_shape=jax.ShapeDtypeStruct((8, D), _f32),
    )(wg, att_s, att_d)


def _gat_prep_body(s_ref, dg_ref, r_ref, wc_ref, tab_ref, ms_ref, md_ref):
    i = pl.program_id(0)
    deg = dg_ref[0, 0, 0, :] + dg_ref[1, 0, 0, :]
    rec = 1.0 / jnp.maximum(deg, 1.0)
    agg = (s_ref[0] + s_ref[1]) * rec[:, None]
    h = jnp.maximum(agg + r_ref[...], 0.0)
    proj = lax.dot_general(h, wc_ref[...], _CONTRACT_11,
                           preferred_element_type=_f32)
    a_s = proj[:, 2]
    a_d = proj[:, 3]
    tab_ref[0, 0, 0, :] = a_s
    tab_ref[1, 0, 0, :] = a_d
    tab_ref[2, 0, 0, :] = proj[:, 0]
    tab_ref[3, 0, 0, :] = proj[:, 1]
    bs = jnp.max(a_s)
    bd = jnp.max(a_d)

    @pl.when(i == 0)
    def _():
        ms_ref[0, 0] = bs
        md_ref[0, 0] = bd

    @pl.when(i > 0)
    def _():
        ms_ref[0, 0] = jnp.maximum(ms_ref[0, 0], bs)
        md_ref[0, 0] = jnp.maximum(md_ref[0, 0], bd)


def _gat_prep(ssum, deg3, r, wc):
    return pl.pallas_call(
        _gat_prep_body,
        grid=(GRID,),
        in_specs=[
            pl.BlockSpec((NC, BLK, D), lambda i: (0, i, 0)),
            pl.BlockSpec((NC, 1, 1, BLK), lambda i: (0, i, 0, 0)),
            pl.BlockSpec((BLK, D), lambda i: (i, 0)),
            pl.BlockSpec((8, D), lambda i: (0, 0)),
        ],
        out_specs=[
            pl.BlockSpec((4, 1, 1, BLK), lambda i: (0, i, 0, 0)),
            pl.BlockSpec(memory_space=pltpu.SMEM),
            pl.BlockSpec(memory_space=pltpu.SMEM),
        ],
        out_shape=[
            jax.ShapeDtypeStruct((4, GRID, 1, BLK), _f32),
            jax.ShapeDtypeStruct((1, 1), _f32),
            jax.ShapeDtypeStruct((1, 1), _f32),
        ],
    )(ssum, deg3, r, wc)


def _final_body(gd_ref, g0_ref, g1_ref, bg_ref, o_ref):
    d = gd_ref[0, 0, :] + gd_ref[1, 0, :]
    n0 = g0_ref[0, 0, :] + g0_ref[1, 0, :]
    n1 = g1_ref[0, 0, :] + g1_ref[1, 0, :]
    rec = 1.0 / (d + 1e-16)
    a0 = n0 * rec + bg_ref[0, 0]
    a1 = n1 * rec + bg_ref[0, 1]
    m = jnp.maximum(a0, a1)
    lse = m + jnp.log(jnp.exp(a0 - m) + jnp.exp(a1 - m))
    o_ref[0, :] = a0 - lse
    o_ref[1, :] = a1 - lse


def _final(gd, gn0, gn1, bg):
    return pl.pallas_call(
        _final_body,
        out_shape=jax.ShapeDtypeStruct((2, N), _f32),
    )(gd, gn0, gn1, bg.reshape(1, 2))


_MESH = plsc.VectorSubcoreMesh(
    core_axis_name="c", subcore_axis_name="s", num_cores=NC, num_subcores=NS)


def _make_seg_kernel(with_deg):
    @functools.partial(
        pl.kernel,
        out_type=(
            jax.ShapeDtypeStruct((NC, N, D), _f32),
            jax.ShapeDtypeStruct((NC, 1, N), _f32),
        ),
        mesh=_MESH,
        compiler_params=pltpu.CompilerParams(needs_layout_passes=False),
        scratch_types=[
            pltpu.VMEM((NG, GSZ), jnp.int32),
            pltpu.VMEM((NG, GSZ), jnp.int32),
            pltpu.VMEM((GSZ, D), _f32),
            pltpu.VMEM((GSZ,), _f32),
            pltpu.VMEM_SHARED((N, D), _f32),
            pltpu.VMEM_SHARED((N,), _f32),
            pltpu.SemaphoreType.DMA,
        ],
    )
    def seg_kernel(tab, src3, dst3, zz, zd, out, deg3,
                   srcv, dstv, rbuf, ones_v, acc, dacc, sem):
        c = lax.axis_index("c")
        s = lax.axis_index("s")
        w = s * NC + c
        pltpu.sync_copy(zz.at[pl.ds(s * RPT, RPT)], acc.at[pl.ds(s * RPT, RPT)])

        @pl.when(s == 0)
        def _():
            pltpu.sync_copy(zz.at[pl.ds(NS * RPT, TAIL)],
                            acc.at[pl.ds(NS * RPT, TAIL)])
            if with_deg:
                pltpu.sync_copy(zd, dacc)

        if with_deg:
            for kk in range(GSZ // 16):
                ones_v[pl.ds(kk * 16, 16)] = jnp.full((16,), 1.0, _f32)
        pltpu.sync_copy(src3.at[w], srcv)
        pltpu.sync_copy(dst3.at[w], dstv)
        plsc.subcore_barrier()

        def body(g, carry):
            pltpu.async_copy(tab.at[srcv.at[g]], rbuf, sem).wait()
            pltpu.sync_copy(rbuf, acc.at[dstv.at[g]], add=True)
            if with_deg:
                pltpu.sync_copy(ones_v, dacc.at[dstv.at[g]], add=True)
            return carry

        lax.fori_loop(0, NG, body, 0)
        plsc.subcore_barrier()
        pltpu.sync_copy(acc.at[pl.ds(s * RPT, RPT)],
                        out.at[c, pl.ds(s * RPT, RPT)])

        @pl.when(s == 0)
        def _():
            pltpu.sync_copy(acc.at[pl.ds(NS * RPT, TAIL)],
                            out.at[c, pl.ds(NS * RPT, TAIL)])
            if with_deg:
                pltpu.sync_copy(dacc, deg3.at[c, 0])

    return seg_kernel


_seg_deg = _make_seg_kernel(True)
_seg_nodeg = _make_seg_kernel(False)


@functools.partial(
    pl.kernel,
    out_type=(
        jax.ShapeDtypeStruct((NC, 1, N), _f32),
        jax.ShapeDtypeStruct((NC, 1, N), _f32),
        jax.ShapeDtypeStruct((NC, 1, N), _f32),
    ),
    mesh=_MESH,
    compiler_params=pltpu.CompilerParams(needs_layout_passes=False),
    scratch_types=[
        pltpu.VMEM((N,), _f32),
        pltpu.VMEM((N,), _f32),
        pltpu.VMEM((N,), _f32),
        pltpu.VMEM((N,), _f32),
        pltpu.VMEM((NG, GSZ), jnp.int32),
        pltpu.VMEM((NG, GSZ), jnp.int32),
        pltpu.VMEM((GSZ,), _f32),
        pltpu.VMEM((GSZ,), _f32),
        pltpu.VMEM((GSZ,), _f32),
        pltpu.VMEM((16,), _f32),
        pltpu.VMEM_SHARED((N,), _f32),
        pltpu.VMEM_SHARED((N,), _f32),
        pltpu.VMEM_SHARED((N,), _f32),
    ],
)
def _gat_edges(tab4, src3, dst3, zd, shift, god, gon0, gon1,
               ast, adt, x0t, x1t, srcv, dstv, exs, w0s, w1s, shv_v,
               dsp, n0sp, n1sp):
    c = lax.axis_index("c")
    s = lax.axis_index("s")
    w = s * NC + c

    @pl.when(s == 0)
    def _():
        pltpu.sync_copy(zd, dsp)
        pltpu.sync_copy(zd, n0sp)
        pltpu.sync_copy(zd, n1sp)

    pltpu.sync_copy(tab4.at[0, 0], ast)
    pltpu.sync_copy(tab4.at[1, 0], adt)
    pltpu.sync_copy(tab4.at[2, 0], x0t)
    pltpu.sync_copy(tab4.at[3, 0], x1t)
    pltpu.sync_copy(shift, shv_v)
    pltpu.sync_copy(src3.at[w], srcv)
    pltpu.sync_copy(dst3.at[w], dstv)
    plsc.subcore_barrier()

    shv = shv_v[...]

    def body(g, carry):
        for kk in range(GSZ // 16):
            sv = srcv[g, pl.ds(kk * 16, 16)]
            dv = dstv[g, pl.ds(kk * 16, 16)]
            asv = plsc.load_gather(ast, [sv])
            adv = plsc.load_gather(adt, [dv])
            e = asv + adv
            e = jnp.maximum(e, 0.2 * e) - shv
            ex = jnp.exp(e)
            x0 = plsc.load_gather(x0t, [sv])
            x1 = plsc.load_gather(x1t, [sv])
            exs[pl.ds(kk * 16, 16)] = ex
            w0s[pl.ds(kk * 16, 16)] = ex * x0
            w1s[pl.ds(kk * 16, 16)] = ex * x1
        pltpu.sync_copy(exs, dsp.at[dstv.at[g]], add=True)
        pltpu.sync_copy(w0s, n0sp.at[dstv.at[g]], add=True)
        pltpu.sync_copy(w1s, n1sp.at[dstv.at[g]], add=True)
        return carry

    lax.fori_loop(0, NG, body, 0)
    plsc.subcore_barrier()

    @pl.when(s == 0)
    def _():
        pltpu.sync_copy(dsp, god.at[c, 0])
        pltpu.sync_copy(n0sp, gon0.at[c, 0])
        pltpu.sync_copy(n1sp, gon1.at[c, 0])


def kernel(x, edge_index, Wl1, Wr1, b1, Wl2, Wr2, b2, Wg, att_src, att_dst, bg):
    src3 = edge_index[0].reshape(NW, NG, GSZ)
    dst3 = edge_index[1].reshape(NW, NG, GSZ)
    zz = jnp.zeros((N, D), _f32)
    zd = jnp.zeros((N,), _f32)

    y1, r1 = _lin2(x, Wl1, Wr1, b1)
    sum1, deg3 = _seg_deg(y1, src3, dst3, zz, zd)

    deg4 = deg3.reshape(NC, GRID, 1, BLK)
    z2, r2 = _sage_next(sum1, deg4, r1, Wl2, Wr2, b2)
    sum2, _ = _seg_nodeg(z2, src3, dst3, zz, zd)

    wc = _wc(Wg, att_src.reshape(1, 2), att_dst.reshape(1, 2))
    tabg, ms, md = _gat_prep(sum2, deg4, r2, wc)
    tab4 = tabg.reshape(4, 1, N)
    m = (ms + md).reshape(())
    shift16 = jnp.broadcast_to(jnp.maximum(m, 0.2 * m), (16,)).astype(_f32)

    god, gon0, gon1 = _gat_edges(tab4, src3, dst3, zd, shift16)

    out2 = _final(god, gon0, gon1, bg)
    return out2.T
